# baseline (speedup 1.0000x reference)
import functools

import jax
import jax.numpy as jnp
from jax import lax
from jax.experimental import pallas as pl
from jax.experimental.pallas import tpu as pltpu
from jax.experimental.pallas import tpu_sc as plsc

N_NODES = 10000
N_EDGES = 320000
D_FEAT = 128
D_EDGE = 4
D_EDGE_PAD = 16

NC = 2
NS = 16
NW = NC * NS

ROWS_PAD = 10112
ROWS_PER_TILE = ROWS_PAD // NS
DUMMY_ROW = 10080
EDGES_PER_WORKER = N_EDGES // NW
MEGA = 400
N_MEGA = EDGES_PER_WORKER // MEGA
SUB = 80
N_SUB = MEGA // SUB
VECS_PER_MEGA = MEGA // 16
RBG = 3
RBE = 2
MASK_WORDS = 320


def _make_sc_kernel():
    mesh = plsc.VectorSubcoreMesh(core_axis_name="c", subcore_axis_name="s")

    @functools.partial(
        pl.kernel,
        out_type=[
            jax.ShapeDtypeStruct((NC, ROWS_PAD, D_FEAT), jnp.float32),
            jax.ShapeDtypeStruct((NC, ROWS_PAD, D_EDGE_PAD), jnp.float32),
        ],
        mesh=mesh,
        compiler_params=pltpu.CompilerParams(
            needs_layout_passes=False, use_tc_tiling_on_sc=False),
        scratch_types=[
            pltpu.VMEM((MASK_WORDS,), jnp.int32),
            pltpu.VMEM((2, MEGA), jnp.int32),
            pltpu.VMEM((2, MEGA), jnp.int32),
            pltpu.VMEM((2, D_EDGE, MEGA), jnp.float32),
            pltpu.VMEM((RBE, SUB, D_EDGE_PAD), jnp.float32),
            pltpu.VMEM((2, N_SUB, SUB), jnp.int32),
            pltpu.VMEM((RBG, SUB, D_FEAT), jnp.float32),
            pltpu.VMEM_SHARED((ROWS_PAD, D_FEAT), jnp.float32),
            pltpu.VMEM_SHARED((ROWS_PAD, D_EDGE_PAD), jnp.float32),
            pltpu.SemaphoreType.DMA((3,)),
            pltpu.SemaphoreType.DMA((RBG,)),
            pltpu.SemaphoreType.DMA((RBG,)),
            pltpu.SemaphoreType.DMA((RBE,)),
        ],
    )
    def sc_kernel(x_hbm, ei_hbm, ea_hbm, mask_hbm, zg_hbm, ze_hbm,
                  g_out, e_out,
                  mask_v, srcb, dstb, eacolb, ea16, effb, rows, g_sh, e_sh,
                  stsem, gsem, sgsem, sesem):
        cid = lax.axis_index("c")
        sid = lax.axis_index("s")
        wid = cid * NS + sid
        r0 = sid * ROWS_PER_TILE

        pltpu.sync_copy(zg_hbm.at[pl.ds(r0, ROWS_PER_TILE)],
                        g_sh.at[pl.ds(r0, ROWS_PER_TILE)])
        pltpu.sync_copy(ze_hbm.at[pl.ds(r0, ROWS_PER_TILE)],
                        e_sh.at[pl.ds(r0, ROWS_PER_TILE)])
        for j in range(RBE):
            pltpu.sync_copy(ze_hbm.at[pl.ds(0, SUB)], ea16.at[j])
        pltpu.sync_copy(mask_hbm, mask_v)

        plsc.subcore_barrier()

        lane = lax.iota(jnp.int32, 16)

        ebase = wid * EDGES_PER_WORKER

        def stage_start(pp, mm):
            b = ebase + mm * MEGA
            return [
                pltpu.make_async_copy(ei_hbm.at[0, pl.ds(b, MEGA)],
                                      srcb.at[pp], stsem.at[0]),
                pltpu.make_async_copy(ei_hbm.at[1, pl.ds(b, MEGA)],
                                      dstb.at[pp], stsem.at[1]),
                pltpu.make_async_copy(ea_hbm.at[:, pl.ds(b, MEGA)],
                                      eacolb.at[pp], stsem.at[2]),
            ]

        def eff_iters(qq, lo, hi):
            def body(i, _):
                sv = srcb[qq, pl.ds(i * 16, 16)]
                dv = dstb[qq, pl.ds(i * 16, 16)]
                ws = plsc.load_gather(mask_v, [sv >> 5])
                wd = plsc.load_gather(mask_v, [dv >> 5])
                keep = ((ws >> (sv & 31)) & (wd >> (dv & 31)) & 1) > 0
                effb[qq, i // 5, pl.ds((i % 5) * 16, 16)] = (
                    jnp.where(keep, dv, DUMMY_ROW))
                return 0
            lax.fori_loop(lo, hi, body, 0)

        for d in stage_start(0, 0):
            d.start()
            d.wait()
        eff_iters(0, 0, VECS_PER_MEGA)

        def mega_body(m, _):
            p = m % 2
            q = 1 - p
            m_next = jnp.minimum(m + 1, N_MEGA - 1)

            stage_descs = stage_start(q, m_next)
            for d in stage_descs:
                d.start()

            def gstart(j):
                d = pltpu.make_async_copy(
                    x_hbm.at[srcb.at[p, pl.ds(j * SUB, SUB)]],
                    rows.at[j % RBG], gsem.at[j % RBG])
                d.start()
                return d

            def widen(j, pp):
                slot = jnp.full((16,), j % RBE, jnp.int32)
                for jc in range(D_EDGE):
                    colv = jnp.full((16,), jc, jnp.int32)
                    for i in range(SUB // 16):
                        v = eacolb[pp, jc, pl.ds(j * SUB + i * 16, 16)]
                        plsc.store_scatter(ea16, [slot, i * 16 + lane, colv], v)

            gd = [None] * N_SUB
            sg = [None] * N_SUB
            se = [None] * N_SUB
            gd[0] = gstart(0)
            gd[1] = gstart(1)
            for k in range(N_SUB):
                if k >= 2:
                    sg[k - 2].wait()
                    se[k - 2].wait()
                if k + 1 < N_SUB:
                    gd[k + 1] = gstart(k + 1)
                widen(k, p)
                gd[k].wait()
                idx = effb.at[p, k]
                sg[k] = pltpu.make_async_copy(rows.at[k % RBG],
                                              g_sh.at[idx], sgsem.at[k % RBG])
                sg[k].start(add=True)
                se[k] = pltpu.make_async_copy(ea16.at[k % RBE],
                                              e_sh.at[idx], sesem.at[k % RBE])
                se[k].start(add=True)
                if k == 1:
                    for d in stage_descs:
                        d.wait()
                elif k >= 2:
                    lo = (k - 2) * 9
                    hi = min(VECS_PER_MEGA, lo + 9)
                    eff_iters(q, lo, hi)
            for k in range(N_SUB - 2, N_SUB):
                sg[k].wait()
                se[k].wait()
            return 0

        lax.fori_loop(0, N_MEGA, mega_body, 0)

        plsc.subcore_barrier()

        pltpu.sync_copy(g_sh.at[pl.ds(r0, ROWS_PER_TILE)],
                        g_out.at[cid, pl.ds(r0, ROWS_PER_TILE)])
        pltpu.sync_copy(e_sh.at[pl.ds(r0, ROWS_PER_TILE)],
                        e_out.at[cid, pl.ds(r0, ROWS_PER_TILE)])

    return sc_kernel


def _dense_body(x_ref, gp_ref, ep_ref, ws_ref, wn_ref, we_ref, b_ref, m_ref, o_ref):
    g = gp_ref[0] + gp_ref[1]
    e = ep_ref[0] + ep_ref[1]
    acc = jnp.dot(x_ref[...], ws_ref[...], preferred_element_type=jnp.float32)
    acc = acc + jnp.dot(g, wn_ref[...], preferred_element_type=jnp.float32)
    acc = acc + jnp.dot(e, we_ref[...], preferred_element_type=jnp.float32)
    acc = acc + b_ref[...]
    o_ref[...] = jnp.maximum(acc, 0.0) * m_ref[...]


_R = 400


def _dense_call(x, gp, ep, W_self, W_nbr, W_edge, b2, m2):
    return pl.pallas_call(
        _dense_body,
        grid=(N_NODES // _R,),
        in_specs=[
            pl.BlockSpec((_R, D_FEAT), lambda i: (i, 0)),
            pl.BlockSpec((NC, _R, D_FEAT), lambda i: (0, i, 0)),
            pl.BlockSpec((NC, _R, D_EDGE_PAD), lambda i: (0, i, 0)),
            pl.BlockSpec((D_FEAT, D_FEAT), lambda i: (0, 0)),
            pl.BlockSpec((D_FEAT, D_FEAT), lambda i: (0, 0)),
            pl.BlockSpec((D_EDGE_PAD, D_FEAT), lambda i: (0, 0)),
            pl.BlockSpec((1, D_FEAT), lambda i: (0, 0)),
            pl.BlockSpec((_R, 1), lambda i: (i, 0)),
        ],
        out_specs=pl.BlockSpec((_R, D_FEAT), lambda i: (i, 0)),
        out_shape=jax.ShapeDtypeStruct((N_NODES, D_FEAT), jnp.float32),
    )(x, gp, ep, W_self, W_nbr, W_edge, b2, m2)


def kernel(x, edge_attr, W_self, W_nbr, W_edge, b, edge_index, mask):
    mp = jnp.pad(mask, (0, MASK_WORDS * 32 - N_NODES))
    mask_bits = (mp.reshape(MASK_WORDS, 32).astype(jnp.uint32)
                 << jnp.arange(32, dtype=jnp.uint32)[None, :]).sum(
                     axis=1).astype(jnp.int32)
    zg = jnp.zeros((ROWS_PAD, D_FEAT), jnp.float32)
    ze = jnp.zeros((ROWS_PAD, D_EDGE_PAD), jnp.float32)
    ea_t = edge_attr.T
    sc = _make_sc_kernel()
    gp, ep = sc(x, edge_index, ea_t, mask_bits, zg, ze)
    b2 = b.reshape(1, D_FEAT)
    m2 = mask.astype(jnp.float32).reshape(N_NODES, 1)
    we_pad = jnp.pad(W_edge, ((0, D_EDGE_PAD - D_EDGE), (0, 0)))
    return _dense_call(x, gp, ep, W_self, W_nbr, we_pad, b2, m2)


# speedup vs baseline: 1.2796x; 1.2796x over previous
import functools

import jax
import jax.numpy as jnp
from jax import lax
from jax.experimental import pallas as pl
from jax.experimental.pallas import tpu as pltpu
from jax.experimental.pallas import tpu_sc as plsc

N_NODES = 10000
N_EDGES = 320000
D_FEAT = 128
D_EDGE = 4
D_EDGE_PAD = 16

NC = 2
NS = 16
NW = NC * NS

ROWS_PAD = 10240
ROWS_PER_TILE = ROWS_PAD // NS
DUMMY_ROW = 10200
EDGES_PER_WORKER = N_EDGES // NW
MEGA = 400
N_MEGA = EDGES_PER_WORKER // MEGA
SUB = 80
N_SUB = MEGA // SUB
VECS_PER_MEGA = MEGA // 16
RBG = 2
RBE = 2
MASK_WORDS = 320


def _make_sc_kernel():
    mesh = plsc.VectorSubcoreMesh(core_axis_name="c", subcore_axis_name="s")

    @functools.partial(
        pl.kernel,
        out_type=[
            jax.ShapeDtypeStruct((NC, ROWS_PAD, D_FEAT), jnp.float32),
            jax.ShapeDtypeStruct((NC, ROWS_PAD, D_EDGE_PAD), jnp.float32),
        ],
        mesh=mesh,
        compiler_params=pltpu.CompilerParams(
            needs_layout_passes=False, use_tc_tiling_on_sc=False),
        scratch_types=[
            pltpu.VMEM((MASK_WORDS,), jnp.int32),
            pltpu.VMEM((2, MEGA), jnp.int32),
            pltpu.VMEM((2, MEGA), jnp.int32),
            pltpu.VMEM((2, D_EDGE, MEGA), jnp.float32),
            pltpu.VMEM((RBE, SUB, D_EDGE_PAD), jnp.float32),
            pltpu.VMEM((2, N_SUB, SUB), jnp.int32),
            pltpu.VMEM((RBG, SUB, D_FEAT), jnp.float32),
            pltpu.VMEM_SHARED((ROWS_PAD, D_FEAT), jnp.float32),
            pltpu.VMEM_SHARED((ROWS_PAD, D_EDGE_PAD), jnp.float32),
            pltpu.SemaphoreType.DMA((3,)),
            pltpu.SemaphoreType.DMA((RBG,)),
            pltpu.SemaphoreType.DMA((RBG,)),
            pltpu.SemaphoreType.DMA((RBE,)),
        ],
    )
    def sc_kernel(x_hbm, ei_hbm, ea_hbm, mask_hbm, zg_hbm, ze_hbm,
                  g_out, e_out,
                  mask_v, srcb, dstb, eacolb, ea16, effb, rows, g_sh, e_sh,
                  stsem, gsem, sgsem, sesem):
        cid = lax.axis_index("c")
        sid = lax.axis_index("s")
        wid = cid * NS + sid
        r0 = sid * ROWS_PER_TILE

        pltpu.sync_copy(zg_hbm.at[pl.ds(r0, ROWS_PER_TILE)],
                        g_sh.at[pl.ds(r0, ROWS_PER_TILE)])
        pltpu.sync_copy(ze_hbm.at[pl.ds(r0, ROWS_PER_TILE)],
                        e_sh.at[pl.ds(r0, ROWS_PER_TILE)])
        for j in range(RBE):
            pltpu.sync_copy(ze_hbm.at[pl.ds(0, SUB)], ea16.at[j])
        pltpu.sync_copy(mask_hbm, mask_v)

        plsc.subcore_barrier()

        lane = lax.iota(jnp.int32, 16)

        ebase = wid * EDGES_PER_WORKER

        def stage_start(pp, mm):
            b = ebase + mm * MEGA
            return [
                pltpu.make_async_copy(ei_hbm.at[0, pl.ds(b, MEGA)],
                                      srcb.at[pp], stsem.at[0]),
                pltpu.make_async_copy(ei_hbm.at[1, pl.ds(b, MEGA)],
                                      dstb.at[pp], stsem.at[1]),
                pltpu.make_async_copy(ea_hbm.at[:, pl.ds(b, MEGA)],
                                      eacolb.at[pp], stsem.at[2]),
            ]

        def eff_iters(qq, lo, hi):
            def body(i, _):
                sv = srcb[qq, pl.ds(i * 16, 16)]
                dv = dstb[qq, pl.ds(i * 16, 16)]
                ws = plsc.load_gather(mask_v, [sv >> 5])
                wd = plsc.load_gather(mask_v, [dv >> 5])
                keep = ((ws >> (sv & 31)) & (wd >> (dv & 31)) & 1) > 0
                effb[qq, i // 5, pl.ds((i % 5) * 16, 16)] = (
                    jnp.where(keep, dv, DUMMY_ROW))
                return 0
            lax.fori_loop(lo, hi, body, 0)

        for d in stage_start(0, 0):
            d.start()
            d.wait()
        eff_iters(0, 0, VECS_PER_MEGA)

        def mega_body(m, _):
            p = m % 2
            q = 1 - p
            m_next = jnp.minimum(m + 1, N_MEGA - 1)

            stage_descs = stage_start(q, m_next)
            for d in stage_descs:
                d.start()

            def gstart(j):
                d = pltpu.make_async_copy(
                    x_hbm.at[srcb.at[p, pl.ds(j * SUB, SUB)]],
                    rows.at[j % RBG], gsem.at[j % RBG])
                d.start()
                return d

            def widen(j, pp):
                slot = jnp.full((16,), j % RBE, jnp.int32)
                for jc in range(D_EDGE):
                    colv = jnp.full((16,), jc, jnp.int32)
                    for i in range(SUB // 16):
                        v = eacolb[pp, jc, pl.ds(j * SUB + i * 16, 16)]
                        plsc.store_scatter(ea16, [slot, i * 16 + lane, colv], v)

            gd = [None] * N_SUB
            sg = [None] * N_SUB
            se = [None] * N_SUB
            gd[0] = gstart(0)
            for k in range(N_SUB):
                if k >= 1:
                    sg[k - 1].wait()
                    se[k - 1].wait()
                if k + 1 < N_SUB:
                    gd[k + 1] = gstart(k + 1)
                widen(k, p)
                gd[k].wait()
                idx = effb.at[p, k]
                sg[k] = pltpu.make_async_copy(rows.at[k % RBG],
                                              g_sh.at[idx], sgsem.at[k % RBG])
                sg[k].start(add=True)
                se[k] = pltpu.make_async_copy(ea16.at[k % RBE],
                                              e_sh.at[idx], sesem.at[k % RBE])
                se[k].start(add=True)
                if k == 1:
                    for d in stage_descs:
                        d.wait()
                elif k >= 2:
                    lo = (k - 2) * 9
                    hi = min(VECS_PER_MEGA, lo + 9)
                    eff_iters(q, lo, hi)
            sg[N_SUB - 1].wait()
            se[N_SUB - 1].wait()
            return 0

        lax.fori_loop(0, N_MEGA, mega_body, 0)

        plsc.subcore_barrier()

        pltpu.sync_copy(g_sh.at[pl.ds(r0, ROWS_PER_TILE)],
                        g_out.at[cid, pl.ds(r0, ROWS_PER_TILE)])
        pltpu.sync_copy(e_sh.at[pl.ds(r0, ROWS_PER_TILE)],
                        e_out.at[cid, pl.ds(r0, ROWS_PER_TILE)])

    return sc_kernel


def _dense_body(x_ref, gp_ref, ep_ref, ws_ref, wn_ref, we_ref, b_ref, m_ref, o_ref):
    g = gp_ref[0] + gp_ref[1]
    e = ep_ref[0] + ep_ref[1]
    acc = jnp.dot(x_ref[...], ws_ref[...], preferred_element_type=jnp.float32)
    acc = acc + jnp.dot(g, wn_ref[...], preferred_element_type=jnp.float32)
    acc = acc + jnp.dot(e, we_ref[...], preferred_element_type=jnp.float32)
    acc = acc + b_ref[...]
    o_ref[...] = jnp.maximum(acc, 0.0) * m_ref[...]


_R = 400


def _dense_call(x, gp, ep, W_self, W_nbr, W_edge, b2, m2):
    return pl.pallas_call(
        _dense_body,
        grid=(N_NODES // _R,),
        in_specs=[
            pl.BlockSpec((_R, D_FEAT), lambda i: (i, 0)),
            pl.BlockSpec((NC, _R, D_FEAT), lambda i: (0, i, 0)),
            pl.BlockSpec((NC, _R, D_EDGE_PAD), lambda i: (0, i, 0)),
            pl.BlockSpec((D_FEAT, D_FEAT), lambda i: (0, 0)),
            pl.BlockSpec((D_FEAT, D_FEAT), lambda i: (0, 0)),
            pl.BlockSpec((D_EDGE_PAD, D_FEAT), lambda i: (0, 0)),
            pl.BlockSpec((1, D_FEAT), lambda i: (0, 0)),
            pl.BlockSpec((_R, 1), lambda i: (i, 0)),
        ],
        out_specs=pl.BlockSpec((_R, D_FEAT), lambda i: (i, 0)),
        out_shape=jax.ShapeDtypeStruct((N_NODES, D_FEAT), jnp.float32),
    )(x, gp, ep, W_self, W_nbr, W_edge, b2, m2)


def kernel(x, edge_attr, W_self, W_nbr, W_edge, b, edge_index, mask):
    mp = jnp.pad(mask, (0, MASK_WORDS * 32 - N_NODES))
    mask_bits = (mp.reshape(MASK_WORDS, 32).astype(jnp.uint32)
                 << jnp.arange(32, dtype=jnp.uint32)[None, :]).sum(
                     axis=1).astype(jnp.int32)
    zg = jnp.zeros((ROWS_PAD, D_FEAT), jnp.float32)
    ze = jnp.zeros((ROWS_PAD, D_EDGE_PAD), jnp.float32)
    ea_t = edge_attr.T
    sc = _make_sc_kernel()
    gp, ep = sc(x, edge_index, ea_t, mask_bits, zg, ze)
    b2 = b.reshape(1, D_FEAT)
    m2 = mask.astype(jnp.float32).reshape(N_NODES, 1)
    we_pad = jnp.pad(W_edge, ((0, D_EDGE_PAD - D_EDGE), (0, 0)))
    return _dense_call(x, gp, ep, W_self, W_nbr, we_pad, b2, m2)
